# baseline (speedup 1.0000x reference)
import functools

import jax
import jax.numpy as jnp
from jax import lax
from jax.experimental import pallas as pl
from jax.experimental.pallas import tpu as pltpu
from jax.experimental.pallas import tpu_sc as plsc

_LANES = 16
_CHUNK = 8


def kernel(inputs, relative_embedding):
    seq_len = inputs.shape[1]
    num_rows, d = relative_embedding.shape
    max_d = (num_rows - 1) // 2
    lo, hi = -seq_len + 1, seq_len - 1

    n_full = num_rows // _CHUNK
    rem = num_rows - n_full * _CHUNK

    dh = d // 2
    dq = d // 4

    mesh = plsc.VectorSubcoreMesh(
        core_axis_name="c", subcore_axis_name="s", num_cores=1
    )

    @functools.partial(
        pl.kernel,
        mesh=mesh,
        out_type=jax.ShapeDtypeStruct((num_rows, d), jnp.float32),
        scratch_types=[
            pltpu.VMEM((_LANES,), jnp.int32),
            pltpu.VMEM((_CHUNK, dq), jnp.float32),
            pltpu.VMEM((_CHUNK, dq), jnp.float32),
            pltpu.SemaphoreType.DMA,
            pltpu.SemaphoreType.DMA,
            pltpu.SemaphoreType.DMA,
        ],
    )
    def emb_gather(table_hbm, out_hbm, idx_v, buf_a, buf_b, sem_a, sem_b, sem_w):
        wid = lax.axis_index("s")
        rc = wid // 2
        coff = (wid % 2) * dh
        base = rc * _CHUNK

        p = lax.iota(jnp.int32, _LANES) + base
        r = jnp.minimum(jnp.maximum(p - max_d, lo), hi) + max_d
        idx_v[...] = jnp.minimum(r, num_rows - 1)
        idx8 = idx_v.at[pl.ds(0, _CHUNK)]

        def run(n_out):
            ga = pltpu.async_copy(
                table_hbm.at[idx8, pl.ds(coff, dq)], buf_a, sem_a
            )
            gb = pltpu.async_copy(
                table_hbm.at[idx8, pl.ds(coff + dq, dq)], buf_b, sem_b
            )
            ga.wait()
            wa = pltpu.async_copy(
                buf_a.at[pl.ds(0, n_out)],
                out_hbm.at[pl.ds(base, n_out), pl.ds(coff, dq)],
                sem_w,
            )
            gb.wait()
            wb = pltpu.async_copy(
                buf_b.at[pl.ds(0, n_out)],
                out_hbm.at[pl.ds(base, n_out), pl.ds(coff + dq, dq)],
                sem_w,
            )
            wa.wait()
            wb.wait()

        @pl.when(rc < n_full)
        def _full():
            run(_CHUNK)

        if rem:

            @pl.when(rc == n_full)
            def _tail():
                run(rem)

    return emb_gather(relative_embedding)


# speedup vs baseline: 11.3257x; 11.3257x over previous
import numpy as np
import jax
import jax.numpy as jnp
from jax.experimental import pallas as pl


def kernel(inputs, relative_embedding):
    seq_len = inputs.shape[1]
    num_rows, d = relative_embedding.shape
    max_d = (num_rows - 1) // 2

    idx = np.arange(-max_d, max_d + 1)
    rel = np.clip(idx, -seq_len + 1, seq_len - 1) + max_d
    runs = []
    start = 0
    for i in range(1, num_rows + 1):
        if i == num_rows or rel[i] != rel[i - 1] + 1:
            runs.append((start, int(rel[start]), i - start))
            start = i

    def body(emb_ref, out_ref):
        for out0, tab0, n in runs:
            out_ref[pl.ds(out0, n), :] = emb_ref[pl.ds(tab0, n), :]

    return pl.pallas_call(
        body,
        out_shape=jax.ShapeDtypeStruct((num_rows, d), jnp.float32),
    )(relative_embedding)
